# baseline (speedup 1.0000x reference)
import functools

import numpy as np

import jax
import jax.numpy as jnp
from jax import lax
from jax.experimental import pallas as pl
from jax.experimental.pallas import tpu as pltpu
from jax.experimental.pallas import tpu_sc as plsc

MAX_LEN = 200
VOCAB = 1000000
DIM = 64
BATCH = 4096

_INFO = plsc.get_sparse_core_info()
NC = _INFO.num_cores
NS = _INFO.num_subcores
LANES = _INFO.num_lanes
NW = NC * NS

BLK = BATCH // NW
DBLKS = DIM // LANES


def _pos_encoding() -> np.ndarray:
    pos = np.arange(MAX_LEN, dtype=np.float64)[:, None]
    i = np.arange(0, DIM, 2, dtype=np.float64)[None, :]
    angle = pos / (10000.0 ** (2.0 * i / DIM))
    enc = np.zeros((MAX_LEN, DIM), dtype=np.float64)
    enc[:, 0::2] = np.sin(angle)
    enc[:, 1::2] = np.cos(angle)
    return enc.astype(np.float32)


def _make_sc_kernel():
    mesh = plsc.VectorSubcoreMesh(core_axis_name="c", subcore_axis_name="s")

    @functools.partial(
        pl.kernel,
        mesh=mesh,
        compiler_params=pltpu.CompilerParams(
            use_tc_tiling_on_sc=False, needs_layout_passes=False
        ),
        out_type=jax.ShapeDtypeStruct((MAX_LEN, DIM, BATCH), jnp.float32),
        scratch_types=[
            pltpu.VMEM((MAX_LEN, BLK), jnp.int32),
            pltpu.VMEM((BLK, DIM), jnp.float32),
            pltpu.VMEM((BLK, DIM), jnp.float32),
            pltpu.VMEM((DIM, BLK), jnp.float32),
            pltpu.VMEM((DIM, BLK), jnp.float32),
            pltpu.VMEM((MAX_LEN * DIM,), jnp.float32),
            pltpu.SemaphoreType.DMA,
            pltpu.SemaphoreType.DMA,
            pltpu.SemaphoreType.DMA,
            pltpu.SemaphoreType.DMA,
        ],
    )
    def k(idx_hbm, table_hbm, pe_hbm, out_hbm,
          idx_v, g0, g1, t0, t1, pe_v, gs0, gs1, os0, os1):
        gbufs, tbufs = (g0, g1), (t0, t1)
        gss, oss = (gs0, gs1), (os0, os1)
        w = lax.axis_index("s") * NC + lax.axis_index("c")
        b0 = w * BLK
        pltpu.sync_copy(idx_hbm.at[w], idx_v)
        pltpu.sync_copy(pe_hbm, pe_v)
        pltpu.async_copy(table_hbm.at[idx_v.at[0]], g0, gs0)
        pltpu.async_copy(table_hbm.at[idx_v.at[1]], g1, gs1)
        d_iota = lax.iota(jnp.int32, LANES)
        zeros16 = jnp.zeros((LANES,), jnp.int32)

        def pair(p, carry):
            for j in range(2):
                l = 2 * p + j

                @pl.when(l >= 2)
                def _():
                    pltpu.make_async_copy(
                        tbufs[j], out_hbm.at[l - 2, :, pl.ds(b0, BLK)], oss[j]
                    ).wait()

                pltpu.make_async_copy(
                    table_hbm.at[idx_v.at[l]], gbufs[j], gss[j]
                ).wait()

                def tr(r, carry2, j=j, l=l):
                    cols = zeros16 + r
                    for db in range(DBLKS):
                        pe_vec = pe_v[pl.ds(l * DIM + db * LANES, LANES)]
                        v = gbufs[j][r, pl.ds(db * LANES, LANES)] + pe_vec
                        plsc.store_scatter(
                            tbufs[j], [d_iota + db * LANES, cols], v
                        )
                    return carry2

                lax.fori_loop(0, BLK, tr, 0, unroll=4)

                @pl.when(l + 2 < MAX_LEN)
                def _():
                    pltpu.async_copy(
                        table_hbm.at[idx_v.at[l + 2]], gbufs[j], gss[j]
                    )

                pltpu.async_copy(
                    tbufs[j], out_hbm.at[l, :, pl.ds(b0, BLK)], oss[j]
                )
            return carry

        lax.fori_loop(0, MAX_LEN // 2, pair, 0)

        pltpu.make_async_copy(
            t0, out_hbm.at[MAX_LEN - 2, :, pl.ds(b0, BLK)], os0
        ).wait()
        pltpu.make_async_copy(
            t1, out_hbm.at[MAX_LEN - 1, :, pl.ds(b0, BLK)], os1
        ).wait()

    return k


_SC_KERNEL = _make_sc_kernel()


def kernel(batch, table):
    idx_t = (
        jnp.transpose(batch.astype(jnp.int32), (1, 0))
        .reshape(MAX_LEN, NW, BLK)
        .transpose(1, 0, 2)
    )
    pe = jnp.asarray(_pos_encoding()).reshape(-1)
    out_t = _SC_KERNEL(idx_t, table, pe)
    return jnp.transpose(out_t, (2, 0, 1))


# speedup vs baseline: 1.0284x; 1.0284x over previous
import functools

import numpy as np

import jax
import jax.numpy as jnp
from jax import lax
from jax.experimental import pallas as pl
from jax.experimental.pallas import tpu as pltpu
from jax.experimental.pallas import tpu_sc as plsc

MAX_LEN = 200
VOCAB = 1000000
DIM = 64
BATCH = 4096

_INFO = plsc.get_sparse_core_info()
NC = _INFO.num_cores
NS = _INFO.num_subcores
LANES = _INFO.num_lanes
NW = NC * NS

BLK = BATCH // NW
DBLKS = DIM // LANES


def _pos_encoding() -> np.ndarray:
    pos = np.arange(MAX_LEN, dtype=np.float64)[:, None]
    i = np.arange(0, DIM, 2, dtype=np.float64)[None, :]
    angle = pos / (10000.0 ** (2.0 * i / DIM))
    enc = np.zeros((MAX_LEN, DIM), dtype=np.float64)
    enc[:, 0::2] = np.sin(angle)
    enc[:, 1::2] = np.cos(angle)
    return enc.astype(np.float32)


def _make_sc_kernel():
    mesh = plsc.VectorSubcoreMesh(core_axis_name="c", subcore_axis_name="s")

    @functools.partial(
        pl.kernel,
        mesh=mesh,
        compiler_params=pltpu.CompilerParams(
            use_tc_tiling_on_sc=False, needs_layout_passes=False
        ),
        out_type=jax.ShapeDtypeStruct((MAX_LEN, DIM, BATCH), jnp.float32),
        scratch_types=[
            pltpu.VMEM((MAX_LEN, BLK), jnp.int32),
            pltpu.VMEM((BLK, DIM), jnp.float32),
            pltpu.VMEM((BLK, DIM), jnp.float32),
            pltpu.VMEM((DIM, BLK), jnp.float32),
            pltpu.VMEM((DIM, BLK), jnp.float32),
            pltpu.VMEM((MAX_LEN * DIM,), jnp.float32),
            pltpu.SemaphoreType.DMA,
            pltpu.SemaphoreType.DMA,
            pltpu.SemaphoreType.DMA,
            pltpu.SemaphoreType.DMA,
        ],
    )
    def k(idx_hbm, table_hbm, pe_hbm, out_hbm,
          idx_v, g0, g1, t0, t1, pe_v, gs0, gs1, os0, os1):
        gbufs, tbufs = (g0, g1), (t0, t1)
        gss, oss = (gs0, gs1), (os0, os1)
        w = lax.axis_index("s") * NC + lax.axis_index("c")
        b0 = w * BLK
        pltpu.sync_copy(idx_hbm.at[w], idx_v)
        pltpu.sync_copy(pe_hbm, pe_v)
        pltpu.async_copy(table_hbm.at[idx_v.at[0]], g0, gs0)
        pltpu.async_copy(table_hbm.at[idx_v.at[1]], g1, gs1)
        d_iota = lax.iota(jnp.int32, LANES)
        zeros16 = jnp.zeros((LANES,), jnp.int32)
        d_rows = [d_iota + db * LANES for db in range(DBLKS)]

        def pair(p, carry):
            for j in range(2):
                l = 2 * p + j

                @pl.when(l >= 2)
                def _():
                    pltpu.make_async_copy(
                        tbufs[j], out_hbm.at[l - 2, :, pl.ds(b0, BLK)], oss[j]
                    ).wait()

                pltpu.make_async_copy(
                    table_hbm.at[idx_v.at[l]], gbufs[j], gss[j]
                ).wait()

                pe_vecs = [
                    pe_v[pl.ds(l * DIM + db * LANES, LANES)]
                    for db in range(DBLKS)
                ]

                def tr(r, carry2, j=j, pe_vecs=pe_vecs):
                    cols = zeros16 + r
                    for db in range(DBLKS):
                        v = gbufs[j][r, pl.ds(db * LANES, LANES)] + pe_vecs[db]
                        plsc.store_scatter(tbufs[j], [d_rows[db], cols], v)
                    return carry2

                lax.fori_loop(0, BLK, tr, 0, unroll=8)

                @pl.when(l + 2 < MAX_LEN)
                def _():
                    pltpu.async_copy(
                        table_hbm.at[idx_v.at[l + 2]], gbufs[j], gss[j]
                    )

                pltpu.async_copy(
                    tbufs[j], out_hbm.at[l, :, pl.ds(b0, BLK)], oss[j]
                )
            return carry

        lax.fori_loop(0, MAX_LEN // 2, pair, 0)

        pltpu.make_async_copy(
            t0, out_hbm.at[MAX_LEN - 2, :, pl.ds(b0, BLK)], os0
        ).wait()
        pltpu.make_async_copy(
            t1, out_hbm.at[MAX_LEN - 1, :, pl.ds(b0, BLK)], os1
        ).wait()

    return k


_SC_KERNEL = _make_sc_kernel()


def kernel(batch, table):
    idx_t = (
        jnp.transpose(batch.astype(jnp.int32), (1, 0))
        .reshape(MAX_LEN, NW, BLK)
        .transpose(1, 0, 2)
    )
    pe = jnp.asarray(_pos_encoding()).reshape(-1)
    out_t = _SC_KERNEL(idx_t, table, pe)
    return jnp.transpose(out_t, (2, 0, 1))


# speedup vs baseline: 1.0601x; 1.0308x over previous
import functools

import numpy as np

import jax
import jax.numpy as jnp
from jax import lax
from jax.experimental import pallas as pl
from jax.experimental.pallas import tpu as pltpu
from jax.experimental.pallas import tpu_sc as plsc

MAX_LEN = 200
VOCAB = 1000000
DIM = 64
BATCH = 4096

_INFO = plsc.get_sparse_core_info()
NC = _INFO.num_cores
NS = _INFO.num_subcores
LANES = _INFO.num_lanes
NW = NC * NS

NBB = 8
BB = BATCH // NBB
NLG = NW // NBB
LPG = MAX_LEN // NLG
QT = 128
NQ = BB // QT
DBLKS = DIM // LANES
PDIM = 2 * DIM


def _pos_encoding() -> np.ndarray:
    pos = np.arange(MAX_LEN, dtype=np.float64)[:, None]
    i = np.arange(0, DIM, 2, dtype=np.float64)[None, :]
    angle = pos / (10000.0 ** (2.0 * i / DIM))
    enc = np.zeros((MAX_LEN, DIM), dtype=np.float64)
    enc[:, 0::2] = np.sin(angle)
    enc[:, 1::2] = np.cos(angle)
    return enc.astype(np.float32)


def _make_sc_kernel():
    mesh = plsc.VectorSubcoreMesh(core_axis_name="c", subcore_axis_name="s")

    @functools.partial(
        pl.kernel,
        mesh=mesh,
        compiler_params=pltpu.CompilerParams(
            use_tc_tiling_on_sc=False, needs_layout_passes=False
        ),
        out_type=jax.ShapeDtypeStruct((MAX_LEN, DIM, BATCH), jnp.float32),
        scratch_types=[
            pltpu.VMEM((LPG, BB), jnp.int32),
            pltpu.VMEM((QT, PDIM), jnp.float32),
            pltpu.VMEM((QT, PDIM), jnp.float32),
            pltpu.VMEM((DIM, BB), jnp.float32),
            pltpu.VMEM((DIM, BB), jnp.float32),
            pltpu.VMEM((LPG * DIM,), jnp.float32),
            pltpu.SemaphoreType.DMA,
            pltpu.SemaphoreType.DMA,
            pltpu.SemaphoreType.DMA,
            pltpu.SemaphoreType.DMA,
        ],
    )
    def k(idx_hbm, table_hbm, pe_hbm, out_hbm,
          idx_v, g0, g1, t0, t1, pe_v, gs0, gs1, os0, os1):
        gbufs, tbufs = (g0, g1), (t0, t1)
        gss, oss = (gs0, gs1), (os0, os1)
        w = lax.axis_index("s") * NC + lax.axis_index("c")
        grp = w // NBB
        b0 = (w % NBB) * BB
        pltpu.sync_copy(idx_hbm.at[w], idx_v)
        pltpu.sync_copy(pe_hbm.at[grp], pe_v)

        def g_start(li, q, qb):
            pltpu.async_copy(
                table_hbm.at[idx_v.at[li, pl.ds(q * QT, QT)]], gbufs[qb],
                gss[qb],
            )

        def g_wait(li, q, qb):
            pltpu.make_async_copy(
                table_hbm.at[idx_v.at[li, pl.ds(q * QT, QT)]], gbufs[qb],
                gss[qb],
            ).wait()

        g_start(0, 0, 0)
        g_start(0, 1, 1)
        d_iota = lax.iota(jnp.int32, LANES)
        zeros16 = jnp.zeros((LANES,), jnp.int32)
        d_rows = [d_iota + db * LANES for db in range(DBLKS)]

        def pair(p, carry):
            for j in range(2):
                li = 2 * p + j
                l = grp * LPG + li

                @pl.when(li >= 2)
                def _():
                    pltpu.make_async_copy(
                        tbufs[j], out_hbm.at[l - 2, :, pl.ds(b0, BB)], oss[j]
                    ).wait()

                pe_vecs = [
                    pe_v[pl.ds(li * DIM + db * LANES, LANES)]
                    for db in range(DBLKS)
                ]

                for q in range(NQ):
                    qb = q % 2
                    g_wait(li, q, qb)

                    def tr(r, carry2, j=j, q=q, qb=qb, pe_vecs=pe_vecs):
                        cols = zeros16 + (q * QT + r)
                        for db in range(DBLKS):
                            v = (gbufs[qb][r, pl.ds(db * LANES, LANES)]
                                 + pe_vecs[db])
                            plsc.store_scatter(
                                tbufs[j], [d_rows[db], cols], v
                            )
                        return carry2

                    lax.fori_loop(0, QT, tr, 0, unroll=8)

                    if q < 2:
                        g_start(li, q + 2, qb)
                    else:
                        @pl.when(li + 1 < LPG)
                        def _(li=li, q=q, qb=qb):
                            g_start(li + 1, q - 2, qb)

                pltpu.async_copy(
                    tbufs[j], out_hbm.at[l, :, pl.ds(b0, BB)], oss[j]
                )
            return carry

        lax.fori_loop(0, LPG // 2, pair, 0)

        last = grp * LPG + LPG
        pltpu.make_async_copy(
            t0, out_hbm.at[last - 2, :, pl.ds(b0, BB)], os0
        ).wait()
        pltpu.make_async_copy(
            t1, out_hbm.at[last - 1, :, pl.ds(b0, BB)], os1
        ).wait()

    return k


_SC_KERNEL = _make_sc_kernel()


def kernel(batch, table):
    idx4 = (
        jnp.transpose(batch.astype(jnp.int32), (1, 0))
        .reshape(NLG, LPG, NBB, BB)
        .transpose(0, 2, 1, 3)
        .reshape(NW, LPG, BB)
    )
    table_p = jnp.pad(table, ((0, 0), (0, PDIM - DIM)))
    pe = jnp.asarray(_pos_encoding()).reshape(NLG, LPG * DIM)
    out_t = _SC_KERNEL(idx4, table_p, pe)
    return jnp.transpose(out_t, (2, 0, 1))


# speedup vs baseline: 1.3346x; 1.2589x over previous
import functools

import numpy as np

import jax
import jax.numpy as jnp
from jax import lax
from jax.experimental import pallas as pl
from jax.experimental.pallas import tpu as pltpu
from jax.experimental.pallas import tpu_sc as plsc

MAX_LEN = 200
VOCAB = 1000000
DIM = 64
BATCH = 4096

_INFO = plsc.get_sparse_core_info()
NC = _INFO.num_cores
NS = _INFO.num_subcores
LANES = _INFO.num_lanes
NW = NC * NS

NBB = 8
BB = BATCH // NBB
NLG = NW // NBB
LPG = MAX_LEN // NLG
QT = 128
NQ = BB // QT
DBLKS = DIM // LANES
PDIM = 2 * DIM


def _pos_encoding() -> np.ndarray:
    pos = np.arange(MAX_LEN, dtype=np.float64)[:, None]
    i = np.arange(0, DIM, 2, dtype=np.float64)[None, :]
    angle = pos / (10000.0 ** (2.0 * i / DIM))
    enc = np.zeros((MAX_LEN, DIM), dtype=np.float64)
    enc[:, 0::2] = np.sin(angle)
    enc[:, 1::2] = np.cos(angle)
    return enc.astype(np.float32)


def _make_sc_kernel():
    mesh = plsc.VectorSubcoreMesh(core_axis_name="c", subcore_axis_name="s")

    @functools.partial(
        pl.kernel,
        mesh=mesh,
        compiler_params=pltpu.CompilerParams(
            use_tc_tiling_on_sc=False, needs_layout_passes=False
        ),
        out_type=jax.ShapeDtypeStruct((MAX_LEN, DIM, BATCH), jnp.float32),
        scratch_types=[
            pltpu.VMEM((LPG, BB), jnp.int32),
            pltpu.VMEM((QT, PDIM), jnp.float32),
            pltpu.VMEM((QT, PDIM), jnp.float32),
            pltpu.VMEM((DIM, BB), jnp.float32),
            pltpu.VMEM((DIM, BB), jnp.float32),
            pltpu.VMEM((LPG * DIM,), jnp.float32),
            pltpu.SemaphoreType.DMA,
            pltpu.SemaphoreType.DMA,
            pltpu.SemaphoreType.DMA,
            pltpu.SemaphoreType.DMA,
        ],
    )
    def k(idx_hbm, table_hbm, pe_hbm, out_hbm,
          idx_v, g0, g1, t0, t1, pe_v, gs0, gs1, os0, os1):
        gbufs, tbufs = (g0, g1), (t0, t1)
        gss, oss = (gs0, gs1), (os0, os1)
        w = lax.axis_index("s") * NC + lax.axis_index("c")
        grp = w // NBB
        b0 = (w % NBB) * BB
        pltpu.sync_copy(idx_hbm.at[w], idx_v)
        pltpu.sync_copy(pe_hbm.at[grp], pe_v)

        def g_start(li, q, qb):
            pltpu.async_copy(
                table_hbm.at[idx_v.at[li, pl.ds(q * QT, QT)]], gbufs[qb],
                gss[qb],
            )

        def g_wait(li, q, qb):
            pltpu.make_async_copy(
                table_hbm.at[idx_v.at[li, pl.ds(q * QT, QT)]], gbufs[qb],
                gss[qb],
            ).wait()

        g_start(0, 0, 0)
        g_start(0, 1, 1)
        d_iota = lax.iota(jnp.int32, LANES)
        zeros16 = jnp.zeros((LANES,), jnp.int32)
        d_rows = [d_iota + db * LANES for db in range(DBLKS)]

        def pair(p, carry):
            for j in range(2):
                li = 2 * p + j
                l = grp * LPG + li

                @pl.when(li >= 2)
                def _():
                    pltpu.make_async_copy(
                        tbufs[j], out_hbm.at[l - 2, :, pl.ds(b0, BB)], oss[j]
                    ).wait()

                pe_vecs = [
                    pe_v[pl.ds(li * DIM + db * LANES, LANES)]
                    for db in range(DBLKS)
                ]

                for q in range(NQ):
                    qb = q % 2
                    g_wait(li, q, qb)

                    @plsc.parallel_loop(0, QT, unroll=8)
                    def _tr(r, j=j, q=q, qb=qb, pe_vecs=pe_vecs):
                        cols = zeros16 + (q * QT + r)
                        for db in range(DBLKS):
                            v = (gbufs[qb][r, pl.ds(db * LANES, LANES)]
                                 + pe_vecs[db])
                            plsc.store_scatter(
                                tbufs[j], [d_rows[db], cols], v
                            )

                    if q < 2:
                        g_start(li, q + 2, qb)
                    else:
                        @pl.when(li + 1 < LPG)
                        def _(li=li, q=q, qb=qb):
                            g_start(li + 1, q - 2, qb)

                pltpu.async_copy(
                    tbufs[j], out_hbm.at[l, :, pl.ds(b0, BB)], oss[j]
                )
            return carry

        lax.fori_loop(0, LPG // 2, pair, 0)

        last = grp * LPG + LPG
        pltpu.make_async_copy(
            t0, out_hbm.at[last - 2, :, pl.ds(b0, BB)], os0
        ).wait()
        pltpu.make_async_copy(
            t1, out_hbm.at[last - 1, :, pl.ds(b0, BB)], os1
        ).wait()

    return k


_SC_KERNEL = _make_sc_kernel()


def kernel(batch, table):
    idx4 = (
        jnp.transpose(batch.astype(jnp.int32), (1, 0))
        .reshape(NLG, LPG, NBB, BB)
        .transpose(0, 2, 1, 3)
        .reshape(NW, LPG, BB)
    )
    table_p = jnp.pad(table, ((0, 0), (0, PDIM - DIM)))
    pe = jnp.asarray(_pos_encoding()).reshape(NLG, LPG * DIM)
    out_t = _SC_KERNEL(idx4, table_p, pe)
    return jnp.transpose(out_t, (2, 0, 1))


# speedup vs baseline: 1.9648x; 1.4722x over previous
import functools

import numpy as np

import jax
import jax.numpy as jnp
from jax import lax
from jax.experimental import pallas as pl
from jax.experimental.pallas import tpu as pltpu
from jax.experimental.pallas import tpu_sc as plsc

MAX_LEN = 200
VOCAB = 1000000
DIM = 64
BATCH = 4096

_INFO = plsc.get_sparse_core_info()
NC = _INFO.num_cores
NS = _INFO.num_subcores
LANES = _INFO.num_lanes
NW = NC * NS

NBB = 8
BB = BATCH // NBB
NLG = NW // NBB
LPG = MAX_LEN // NLG
QT = 128
NQ = BB // QT
DBLKS = DIM // LANES
PDIM = 2 * DIM


def _pos_encoding() -> np.ndarray:
    pos = np.arange(MAX_LEN, dtype=np.float64)[:, None]
    i = np.arange(0, DIM, 2, dtype=np.float64)[None, :]
    angle = pos / (10000.0 ** (2.0 * i / DIM))
    enc = np.zeros((MAX_LEN, DIM), dtype=np.float64)
    enc[:, 0::2] = np.sin(angle)
    enc[:, 1::2] = np.cos(angle)
    return enc.astype(np.float32)


def _make_sc_kernel():
    mesh = plsc.VectorSubcoreMesh(core_axis_name="c", subcore_axis_name="s")

    @functools.partial(
        pl.kernel,
        mesh=mesh,
        compiler_params=pltpu.CompilerParams(
            use_tc_tiling_on_sc=False, needs_layout_passes=False
        ),
        out_type=jax.ShapeDtypeStruct((MAX_LEN, DIM, BATCH), jnp.float32),
        scratch_types=[
            pltpu.VMEM((LPG, BB), jnp.int32),
            pltpu.VMEM((QT, PDIM), jnp.float32),
            pltpu.VMEM((QT, PDIM), jnp.float32),
            pltpu.VMEM((DIM, BB + 1), jnp.float32),
            pltpu.VMEM((DIM, BB + 1), jnp.float32),
            pltpu.VMEM((LPG * DIM,), jnp.float32),
            pltpu.SemaphoreType.DMA,
            pltpu.SemaphoreType.DMA,
            pltpu.SemaphoreType.DMA,
            pltpu.SemaphoreType.DMA,
        ],
    )
    def k(idx_hbm, table_hbm, pe_hbm, out_hbm,
          idx_v, g0, g1, t0, t1, pe_v, gs0, gs1, os0, os1):
        gbufs, tbufs = (g0, g1), (t0, t1)
        gss, oss = (gs0, gs1), (os0, os1)
        w = lax.axis_index("s") * NC + lax.axis_index("c")
        grp = w // NBB
        b0 = (w % NBB) * BB
        pltpu.sync_copy(idx_hbm.at[w], idx_v)
        pltpu.sync_copy(pe_hbm.at[grp], pe_v)

        def g_start(li, q, qb):
            pltpu.async_copy(
                table_hbm.at[idx_v.at[li, pl.ds(q * QT, QT)]],
                gbufs[qb], gss[qb],
            )

        def g_wait(li, q, qb):
            pltpu.make_async_copy(
                table_hbm.at[idx_v.at[li, pl.ds(q * QT, QT)]],
                gbufs[qb], gss[qb],
            ).wait()

        g_start(0, 0, 0)
        g_start(0, 1, 1)
        d_iota = lax.iota(jnp.int32, LANES)
        zeros16 = jnp.zeros((LANES,), jnp.int32)
        d_rows = [d_iota + db * LANES for db in range(DBLKS)]

        def pair(p, carry):
            for j in range(2):
                li = 2 * p + j
                l = grp * LPG + li

                @pl.when(li >= 2)
                def _():
                    pltpu.make_async_copy(
                        tbufs[j].at[:, pl.ds(0, BB)],
                        out_hbm.at[l - 2, :, pl.ds(b0, BB)], oss[j]
                    ).wait()

                pe_vecs = [
                    pe_v[pl.ds(li * DIM + db * LANES, LANES)]
                    for db in range(DBLKS)
                ]

                for q in range(NQ):
                    qb = q % 2
                    g_wait(li, q, qb)

                    @plsc.parallel_loop(0, QT, unroll=8)
                    def _tr(r, j=j, q=q, qb=qb, pe_vecs=pe_vecs):
                        cols = zeros16 + (q * QT + r)
                        for db in range(DBLKS):
                            v = (gbufs[qb][r, pl.ds(db * LANES, LANES)]
                                 + pe_vecs[db])
                            plsc.store_scatter(
                                tbufs[j], [d_rows[db], cols], v
                            )

                    if q < 2:
                        g_start(li, q + 2, qb)
                    else:
                        @pl.when(li + 1 < LPG)
                        def _(li=li, q=q, qb=qb):
                            g_start(li + 1, q - 2, qb)

                pltpu.async_copy(
                    tbufs[j].at[:, pl.ds(0, BB)],
                    out_hbm.at[l, :, pl.ds(b0, BB)], oss[j]
                )
            return carry

        lax.fori_loop(0, LPG // 2, pair, 0)

        last = grp * LPG + LPG
        pltpu.make_async_copy(
            t0.at[:, pl.ds(0, BB)], out_hbm.at[last - 2, :, pl.ds(b0, BB)], os0
        ).wait()
        pltpu.make_async_copy(
            t1.at[:, pl.ds(0, BB)], out_hbm.at[last - 1, :, pl.ds(b0, BB)], os1
        ).wait()

    return k


_SC_KERNEL = _make_sc_kernel()


def kernel(batch, table):
    idx4 = (
        jnp.transpose(batch.astype(jnp.int32), (1, 0))
        .reshape(NLG, LPG, NBB, BB)
        .transpose(0, 2, 1, 3)
        .reshape(NW, LPG, BB)
    )
    table_p = jnp.pad(table, ((0, 0), (0, PDIM - DIM)))
    pe = jnp.asarray(_pos_encoding()).reshape(NLG, LPG * DIM)
    out_t = _SC_KERNEL(idx4, table_p, pe)
    return jnp.transpose(out_t, (2, 0, 1))
